# Initial kernel scaffold; baseline (speedup 1.0000x reference)
import jax
import jax.numpy as jnp
from jax.experimental import pallas as pl


def kernel(x, edge_index, W0, b0, W1, b1, W2, b2, bn0_w, bn0_b, bn0_m, bn0_v, bn1_w, bn1_b, bn1_m, bn1_v, bn2_w, bn2_b, bn2_m, bn2_v, hW0, hb0, hW1, hb1):
    raise NotImplementedError("write your pallas kernel here")


# speedup vs baseline: 7.0935x; 7.0935x over previous
import functools
import jax
import jax.numpy as jnp
from jax import lax
from jax.experimental import pallas as pl
from jax.experimental.pallas import tpu as pltpu
from jax.experimental.pallas import tpu_sc as plsc

N = 10000
E = 320000
D_IN = 128
D_H = 256
D_OUT = 128
EPS = 1e-5

NC = 2
NS = 16
C = 128

EPAD = ((E + NC * NS * C - 1) // (NC * NS * C)) * (NC * NS * C)
EPT_AGG = EPAD // NS
EPT_DEG = EPAD // (NC * NS)
NACC = 10240
NDEG = 10240
RB = 1000


def _build_sc_kernels():
    mesh = plsc.VectorSubcoreMesh(core_axis_name="c", subcore_axis_name="s",
                                  num_cores=NC, num_subcores=NS)

    @functools.partial(
        pl.kernel,
        out_type=jax.ShapeDtypeStruct((NC, NDEG), jnp.float32),
        mesh=mesh,
        scratch_types=[
            pltpu.VMEM((C,), jnp.int32),
            pltpu.VMEM((C,), jnp.float32),
            pltpu.VMEM_SHARED((NDEG,), jnp.float32),
        ],
    )
    def deg_kernel(dst_hbm, zeros1_hbm, out_hbm, dst_v, ones_v, deg_sh):
        c = lax.axis_index("c")
        s = lax.axis_index("s")
        stripe = NDEG // NS
        pltpu.sync_copy(zeros1_hbm.at[pl.ds(0, stripe)],
                        deg_sh.at[pl.ds(s * stripe, stripe)])
        for j in range(C // 16):
            ones_v[pl.ds(j * 16, 16)] = jnp.full((16,), 1.0, jnp.float32)
        plsc.subcore_barrier()

        def body(k, carry):
            off = (c * NS + s) * EPT_DEG + k * C
            pltpu.sync_copy(dst_hbm.at[pl.ds(off, C)], dst_v)
            pltpu.sync_copy(ones_v, deg_sh.at[dst_v], add=True)
            return carry

        lax.fori_loop(0, EPT_DEG // C, body, 0)
        plsc.subcore_barrier()
        pltpu.sync_copy(deg_sh.at[pl.ds(s * stripe, stripe)],
                        out_hbm.at[c].at[pl.ds(s * stripe, stripe)])

    @functools.partial(
        pl.kernel,
        out_type=jax.ShapeDtypeStruct((NC * NACC, 128), jnp.float32),
        mesh=mesh,
        scratch_types=[
            pltpu.VMEM((C,), jnp.int32),
            pltpu.VMEM((C,), jnp.int32),
            pltpu.VMEM((C, 128), jnp.float32),
            pltpu.VMEM_SHARED((NACC, 128), jnp.float32),
            pltpu.SemaphoreType.DMA,
        ],
    )
    def agg_kernel(h_hbm, src_hbm, dst_hbm, zeros_hbm, out_hbm,
                   src_v, dst_v, rows_v, acc_sh, sem):
        c = lax.axis_index("c")
        s = lax.axis_index("s")
        zrows = NACC // NS
        pltpu.sync_copy(zeros_hbm.at[pl.ds(0, zrows)],
                        acc_sh.at[pl.ds(s * zrows, zrows)])
        plsc.subcore_barrier()
        cbase = c * NACC

        def body(k, carry):
            off = s * EPT_AGG + k * C
            pltpu.sync_copy(src_hbm.at[pl.ds(off, C)], src_v)
            pltpu.sync_copy(dst_hbm.at[pl.ds(off, C)], dst_v)
            for j in range(C // 16):
                src_v[pl.ds(j * 16, 16)] = src_v[pl.ds(j * 16, 16)] + cbase
            pltpu.async_copy(h_hbm.at[src_v], rows_v, sem).wait()
            pltpu.sync_copy(rows_v, acc_sh.at[dst_v], add=True)
            return carry

        lax.fori_loop(0, EPT_AGG // C, body, 0)
        plsc.subcore_barrier()
        drows = NACC // NS
        pltpu.sync_copy(acc_sh.at[pl.ds(s * drows, drows)],
                        out_hbm.at[pl.ds(c * NACC + s * drows, drows)])

    return deg_kernel, agg_kernel


def _dinv_body(degp_ref, o_ref):
    deg = jnp.sum(degp_ref[...], axis=0) + 1.0
    o_ref[...] = lax.rsqrt(deg)[:, None]


def _dinv_prep(degp):
    blk = NDEG // 8
    return pl.pallas_call(
        _dinv_body,
        grid=(8,),
        in_specs=[pl.BlockSpec((NC, blk), lambda i: (0, i))],
        out_specs=pl.BlockSpec((blk, 1), lambda i: (i, 0)),
        out_shape=jax.ShapeDtypeStruct((NDEG, 1), jnp.float32),
    )(degp)


def _mm_body(x_ref, w_ref, dinv_ref, o_ref):
    h = jnp.dot(x_ref[...], w_ref[...], preferred_element_type=jnp.float32)
    o_ref[...] = (h * dinv_ref[...])[None]


def _mm_scale_split(x, w, dinv):
    din = x.shape[1]
    return pl.pallas_call(
        _mm_body,
        grid=(10, NC),
        in_specs=[
            pl.BlockSpec((RB, din), lambda i, c: (i, 0)),
            pl.BlockSpec((din, 128), lambda i, c: (0, c)),
            pl.BlockSpec((RB, 1), lambda i, c: (i, 0)),
        ],
        out_specs=pl.BlockSpec((1, RB, 128), lambda i, c: (c, i, 0)),
        out_shape=jax.ShapeDtypeStruct((NC, NACC, 128), jnp.float32),
    )(x, w, dinv)


def _comb_body(agg_ref, h_ref, dinv_ref, sc_ref, sh_ref, o_ref):
    z = dinv_ref[...] * (agg_ref[0] + h_ref[0])
    o_ref[...] = jnp.maximum(z * sc_ref[...] + sh_ref[...], 0.0)


def _comb_res_body(agg_ref, h_ref, dinv_ref, sc_ref, sh_ref, r_ref, o_ref):
    z = dinv_ref[...] * (agg_ref[0] + h_ref[0])
    o_ref[...] = jnp.maximum(z * sc_ref[...] + sh_ref[...], 0.0) + r_ref[...]


def _combine(agg2, h2, dinv, scale, shift, res=None):
    in_specs = [
        pl.BlockSpec((1, RB, 128), lambda i, c: (c, i, 0)),
        pl.BlockSpec((1, RB, 128), lambda i, c: (c, i, 0)),
        pl.BlockSpec((RB, 1), lambda i, c: (i, 0)),
        pl.BlockSpec((1, 128), lambda i, c: (0, c)),
        pl.BlockSpec((1, 128), lambda i, c: (0, c)),
    ]
    args = [agg2, h2, dinv, scale, shift]
    body = _comb_body
    if res is not None:
        in_specs.append(pl.BlockSpec((RB, 128), lambda i, c: (i, c)))
        args.append(res)
        body = _comb_res_body
    return pl.pallas_call(
        body,
        grid=(10, NC),
        in_specs=in_specs,
        out_specs=pl.BlockSpec((RB, 128), lambda i, c: (i, c)),
        out_shape=jax.ShapeDtypeStruct((N, D_H), jnp.float32),
    )(*args)


def _head_body(x_ref, w0_ref, b0_ref, w1_ref, b1_ref, o_ref):
    t = jnp.dot(x_ref[...], w0_ref[...], preferred_element_type=jnp.float32)
    t = jnp.maximum(t + b0_ref[...], 0.0)
    o_ref[...] = jnp.dot(t, w1_ref[...],
                         preferred_element_type=jnp.float32) + b1_ref[...]


def _head(x, hW0, hb0, hW1, hb1):
    return pl.pallas_call(
        _head_body,
        grid=(10,),
        in_specs=[
            pl.BlockSpec((RB, D_H), lambda i: (i, 0)),
            pl.BlockSpec((D_H, D_H), lambda i: (0, 0)),
            pl.BlockSpec((1, D_H), lambda i: (0, 0)),
            pl.BlockSpec((D_H, D_OUT), lambda i: (0, 0)),
            pl.BlockSpec((1, D_OUT), lambda i: (0, 0)),
        ],
        out_specs=pl.BlockSpec((RB, D_OUT), lambda i: (i, 0)),
        out_shape=jax.ShapeDtypeStruct((N, D_OUT), jnp.float32),
    )(x, hW0, hb0.reshape(1, D_H), hW1, hb1.reshape(1, D_OUT))


def kernel(x, edge_index, W0, b0, W1, b1, W2, b2,
           bn0_w, bn0_b, bn0_m, bn0_v,
           bn1_w, bn1_b, bn1_m, bn1_v,
           bn2_w, bn2_b, bn2_m, bn2_v,
           hW0, hb0, hW1, hb1):
    deg_kernel, agg_kernel = _build_sc_kernels()

    pad = EPAD - E
    srcp = jnp.concatenate([edge_index[0],
                            jnp.zeros((pad,), jnp.int32)])
    dstp = jnp.concatenate([edge_index[1],
                            jnp.full((pad,), N, jnp.int32)])
    zeros1 = jnp.zeros((NDEG // NS,), jnp.float32)
    zeros2 = jnp.zeros((NACC // NS, 128), jnp.float32)

    degp = deg_kernel(dstp, zeros1)
    dinv_full = _dinv_prep(degp)
    dinv = dinv_full[:N]

    convs = [(W0, b0), (W1, b1), (W2, b2)]
    bns = [(bn0_w, bn0_b, bn0_m, bn0_v),
           (bn1_w, bn1_b, bn1_m, bn1_v),
           (bn2_w, bn2_b, bn2_m, bn2_v)]

    x_cur = x
    for i in range(3):
        W, b = convs[i]
        bw, bb, bm, bv = bns[i]
        scale = (bw * lax.rsqrt(bv + EPS)).reshape(1, D_H)
        shift = ((b - bm) * scale[0] + bb).reshape(1, D_H)
        h2 = _mm_scale_split(x_cur, W, dinv)
        agg_flat = agg_kernel(h2.reshape(NC * NACC, 128), srcp, dstp, zeros2)
        agg2 = agg_flat.reshape(NC, NACC, 128)
        res = x_cur if i > 0 else None
        x_cur = _combine(agg2, h2, dinv, scale, shift, res)

    return _head(x_cur, hW0, hb0, hW1, hb1)


# speedup vs baseline: 7.2227x; 1.0182x over previous
import functools
import jax
import jax.numpy as jnp
from jax import lax
from jax.experimental import pallas as pl
from jax.experimental.pallas import tpu as pltpu
from jax.experimental.pallas import tpu_sc as plsc

N = 10000
E = 320000
D_IN = 128
D_H = 256
D_OUT = 128
EPS = 1e-5

NC = 2
NS = 16
C = 128

EPAD = 327680
NCH = EPAD // C
CH_AGG = NCH // NS
CH_DEG = NCH // (NC * NS)
NACC = 10240
NDEG = 10240
RB = 1000


def _build_sc_kernels():
    mesh = plsc.VectorSubcoreMesh(core_axis_name="c", subcore_axis_name="s",
                                  num_cores=NC, num_subcores=NS)

    @functools.partial(
        pl.kernel,
        out_type=jax.ShapeDtypeStruct((NC, NDEG), jnp.float32),
        mesh=mesh,
        scratch_types=[
            pltpu.VMEM((CH_DEG, C), jnp.int32),
            pltpu.VMEM((C,), jnp.float32),
            pltpu.VMEM_SHARED((NDEG,), jnp.float32),
        ],
    )
    def deg_kernel(dst_hbm, zeros1_hbm, out_hbm, dst_v, ones_v, deg_sh):
        c = lax.axis_index("c")
        s = lax.axis_index("s")
        stripe = NDEG // NS
        pltpu.sync_copy(zeros1_hbm.at[pl.ds(0, stripe)],
                        deg_sh.at[pl.ds(s * stripe, stripe)])
        pltpu.sync_copy(dst_hbm.at[pl.ds((c * NS + s) * CH_DEG, CH_DEG)], dst_v)
        for j in range(C // 16):
            ones_v[pl.ds(j * 16, 16)] = jnp.full((16,), 1.0, jnp.float32)
        plsc.subcore_barrier()

        def body(k, carry):
            pltpu.sync_copy(ones_v, deg_sh.at[dst_v.at[k]], add=True)
            return carry

        lax.fori_loop(0, CH_DEG, body, 0)
        plsc.subcore_barrier()
        pltpu.sync_copy(deg_sh.at[pl.ds(s * stripe, stripe)],
                        out_hbm.at[c].at[pl.ds(s * stripe, stripe)])

    B = 16
    NBLK = CH_AGG // B

    @functools.partial(
        pl.kernel,
        out_type=jax.ShapeDtypeStruct((NC * NACC, 128), jnp.float32),
        mesh=mesh,
        scratch_types=[
            pltpu.VMEM((B, C), jnp.int32),
            pltpu.VMEM((B, C), jnp.int32),
            pltpu.VMEM((B, C), jnp.int32),
            pltpu.VMEM((B, C), jnp.int32),
            pltpu.VMEM((C, 128), jnp.float32),
            pltpu.VMEM((C, 128), jnp.float32),
            pltpu.VMEM_SHARED((NACC, 128), jnp.float32),
            pltpu.SemaphoreType.DMA,
            pltpu.SemaphoreType.DMA,
            pltpu.SemaphoreType.DMA,
            pltpu.SemaphoreType.DMA,
        ],
    )
    def agg_kernel(h_hbm, src_hbm, dst_hbm, zeros_hbm, out_hbm,
                   sb0, sb1, db0, db1, rows0, rows1, acc_sh,
                   isem0, isem1, gsem0, gsem1):
        c = lax.axis_index("c")
        s = lax.axis_index("s")
        zrows = NACC // NS
        pltpu.sync_copy(zeros_hbm.at[pl.ds(0, zrows)],
                        acc_sh.at[pl.ds(s * zrows, zrows)])
        srow0 = c * NCH + s * CH_AGG
        drow0 = s * CH_AGG
        rbufs = (rows0, rows1)
        gsems = (gsem0, gsem1)

        def issue_idx(bi, sb, db, isem):
            pltpu.async_copy(src_hbm.at[pl.ds(srow0 + bi * B, B)], sb, isem)
            pltpu.async_copy(dst_hbm.at[pl.ds(drow0 + bi * B, B)], db, isem)

        def process_block(bi, sb, db, isem):
            pltpu.make_async_copy(src_hbm.at[pl.ds(srow0, B)], sb, isem).wait()
            pltpu.make_async_copy(dst_hbm.at[pl.ds(drow0, B)], db, isem).wait()
            pltpu.async_copy(h_hbm.at[sb.at[0]], rows0, gsem0)
            pltpu.async_copy(h_hbm.at[sb.at[1]], rows1, gsem1)
            for kk in range(B):
                rb = rbufs[kk % 2]
                gs = gsems[kk % 2]
                pltpu.make_async_copy(h_hbm.at[sb.at[kk]], rb, gs).wait()
                pltpu.sync_copy(rb, acc_sh.at[db.at[kk]], add=True)
                if kk + 2 < B:
                    pltpu.async_copy(h_hbm.at[sb.at[kk + 2]], rb, gs)
            @pl.when(bi + 2 < NBLK)
            def _():
                issue_idx(bi + 2, sb, db, isem)

        plsc.subcore_barrier()
        issue_idx(0, sb0, db0, isem0)
        issue_idx(1, sb1, db1, isem1)

        def superbody(u, carry):
            process_block(2 * u, sb0, db0, isem0)
            process_block(2 * u + 1, sb1, db1, isem1)
            return carry

        lax.fori_loop(0, NBLK // 2, superbody, 0)
        plsc.subcore_barrier()
        drows = NACC // NS
        pltpu.sync_copy(acc_sh.at[pl.ds(s * drows, drows)],
                        out_hbm.at[pl.ds(c * NACC + s * drows, drows)])

    return deg_kernel, agg_kernel


def _dinv_body(degp_ref, o_ref):
    deg = jnp.sum(degp_ref[...], axis=0) + 1.0
    o_ref[...] = lax.rsqrt(deg)[:, None]


def _dinv_prep(degp):
    blk = NDEG // 8
    return pl.pallas_call(
        _dinv_body,
        grid=(8,),
        in_specs=[pl.BlockSpec((NC, blk), lambda i: (0, i))],
        out_specs=pl.BlockSpec((blk, 1), lambda i: (i, 0)),
        out_shape=jax.ShapeDtypeStruct((NDEG, 1), jnp.float32),
    )(degp)


def _mm_body(x_ref, w_ref, dinv_ref, o_ref):
    h = jnp.dot(x_ref[...], w_ref[...], preferred_element_type=jnp.float32)
    o_ref[...] = (h * dinv_ref[...])[None]


def _mm_scale_split(x, w, dinv):
    din = x.shape[1]
    return pl.pallas_call(
        _mm_body,
        grid=(10, NC),
        in_specs=[
            pl.BlockSpec((RB, din), lambda i, c: (i, 0)),
            pl.BlockSpec((din, 128), lambda i, c: (0, c)),
            pl.BlockSpec((RB, 1), lambda i, c: (i, 0)),
        ],
        out_specs=pl.BlockSpec((1, RB, 128), lambda i, c: (c, i, 0)),
        out_shape=jax.ShapeDtypeStruct((NC, NACC, 128), jnp.float32),
    )(x, w, dinv)


def _comb_body(agg_ref, h_ref, dinv_ref, sc_ref, sh_ref, o_ref):
    z = dinv_ref[...] * (agg_ref[0] + h_ref[0])
    o_ref[...] = jnp.maximum(z * sc_ref[...] + sh_ref[...], 0.0)


def _comb_res_body(agg_ref, h_ref, dinv_ref, sc_ref, sh_ref, r_ref, o_ref):
    z = dinv_ref[...] * (agg_ref[0] + h_ref[0])
    o_ref[...] = jnp.maximum(z * sc_ref[...] + sh_ref[...], 0.0) + r_ref[...]


def _combine(agg2, h2, dinv, scale, shift, res=None):
    in_specs = [
        pl.BlockSpec((1, RB, 128), lambda i, c: (c, i, 0)),
        pl.BlockSpec((1, RB, 128), lambda i, c: (c, i, 0)),
        pl.BlockSpec((RB, 1), lambda i, c: (i, 0)),
        pl.BlockSpec((1, 128), lambda i, c: (0, c)),
        pl.BlockSpec((1, 128), lambda i, c: (0, c)),
    ]
    args = [agg2, h2, dinv, scale, shift]
    body = _comb_body
    if res is not None:
        in_specs.append(pl.BlockSpec((RB, 128), lambda i, c: (i, c)))
        args.append(res)
        body = _comb_res_body
    return pl.pallas_call(
        body,
        grid=(10, NC),
        in_specs=in_specs,
        out_specs=pl.BlockSpec((RB, 128), lambda i, c: (i, c)),
        out_shape=jax.ShapeDtypeStruct((N, D_H), jnp.float32),
    )(*args)


def _head_body(x_ref, w0_ref, b0_ref, w1_ref, b1_ref, o_ref):
    t = jnp.dot(x_ref[...], w0_ref[...], preferred_element_type=jnp.float32)
    t = jnp.maximum(t + b0_ref[...], 0.0)
    o_ref[...] = jnp.dot(t, w1_ref[...],
                         preferred_element_type=jnp.float32) + b1_ref[...]


def _head(x, hW0, hb0, hW1, hb1):
    return pl.pallas_call(
        _head_body,
        grid=(10,),
        in_specs=[
            pl.BlockSpec((RB, D_H), lambda i: (i, 0)),
            pl.BlockSpec((D_H, D_H), lambda i: (0, 0)),
            pl.BlockSpec((1, D_H), lambda i: (0, 0)),
            pl.BlockSpec((D_H, D_OUT), lambda i: (0, 0)),
            pl.BlockSpec((1, D_OUT), lambda i: (0, 0)),
        ],
        out_specs=pl.BlockSpec((RB, D_OUT), lambda i: (i, 0)),
        out_shape=jax.ShapeDtypeStruct((N, D_OUT), jnp.float32),
    )(x, hW0, hb0.reshape(1, D_H), hW1, hb1.reshape(1, D_OUT))


def kernel(x, edge_index, W0, b0, W1, b1, W2, b2,
           bn0_w, bn0_b, bn0_m, bn0_v,
           bn1_w, bn1_b, bn1_m, bn1_v,
           bn2_w, bn2_b, bn2_m, bn2_v,
           hW0, hb0, hW1, hb1):
    deg_kernel, agg_kernel = _build_sc_kernels()

    pad = EPAD - E
    srcp = jnp.concatenate([edge_index[0],
                            jnp.zeros((pad,), jnp.int32)]).reshape(NCH, C)
    dstp = jnp.concatenate([edge_index[1],
                            jnp.full((pad,), N, jnp.int32)]).reshape(NCH, C)
    src2 = jnp.concatenate([srcp, srcp + NACC], axis=0)
    zeros1 = jnp.zeros((NDEG // NS,), jnp.float32)
    zeros2 = jnp.zeros((NACC // NS, 128), jnp.float32)

    degp = deg_kernel(dstp, zeros1)
    dinv_full = _dinv_prep(degp)
    dinv = dinv_full[:N]

    convs = [(W0, b0), (W1, b1), (W2, b2)]
    bns = [(bn0_w, bn0_b, bn0_m, bn0_v),
           (bn1_w, bn1_b, bn1_m, bn1_v),
           (bn2_w, bn2_b, bn2_m, bn2_v)]

    x_cur = x
    for i in range(3):
        W, b = convs[i]
        bw, bb, bm, bv = bns[i]
        scale = (bw * lax.rsqrt(bv + EPS)).reshape(1, D_H)
        shift = ((b - bm) * scale[0] + bb).reshape(1, D_H)
        h2 = _mm_scale_split(x_cur, W, dinv)
        agg_flat = agg_kernel(h2.reshape(NC * NACC, 128), src2, dstp, zeros2)
        agg2 = agg_flat.reshape(NC, NACC, 128)
        res = x_cur if i > 0 else None
        x_cur = _combine(agg2, h2, dinv, scale, shift, res)

    return _head(x_cur, hW0, hb0, hW1, hb1)
